# baseline (speedup 1.0000x reference)
import jax
import jax.numpy as jnp
import numpy as np
from jax.experimental import pallas as pl
from jax.experimental.pallas import tpu as pltpu

_PH = jax.lax.Precision.HIGHEST

NLP = 2048
NSP = 8192
TL = 256
CS = 1024
NSC = NSP // CS
HID = 128
R2 = 3.5 * 3.5
BC = 8.0

_R_W1 = 0
_R_W2T = 512
_R_GWT = 1024
_R_BWT = 1152
_R_GCNT = 1280
_R_CW4 = 1408
_R_BSW4 = 1412
_R_BSB = 1416
_R_B2 = 1417
_R_GB2 = 1418
_R_CB = 1419
_R_GB = 1420
_R_WP = 1421
_R_BP = 1424
_PACK_ROWS = 1432


def _mmT(x, w):
    return jax.lax.dot_general(x, w, (((1,), (1,)), ((), ())),
                               preferred_element_type=jnp.float32)


def _mm(x, w):
    return jax.lax.dot_general(x, w, (((1,), (0,)), ((), ())),
                               preferred_element_type=jnp.float32)


def _fused_kernel(bounds_ref,
                  posl4_ref, sql_ref, t_ref,
                  poss4_ref, sqs_ref,
                  pack_ref, b1_ref,
                  pred_ref, hs_ref, hl_ref, acc_ref, deg_ref):
    i = pl.program_id(0)

    @pl.when(i == 0)
    def _features():
        hs_ref[...] = (_mm(poss4_ref[...], pack_ref[_R_BSW4:_R_BSW4 + 4, :])
                       + pack_ref[_R_BSB:_R_BSB + 1, :])
        half = HID // 2
        emb = np.log(10000.0) / (half - 1)
        k = jax.lax.broadcasted_iota(jnp.int32, (1, half), 1).astype(
            jnp.float32)
        freqs = jnp.exp(k * (-emb))
        args = t_ref[...] * freqs
        temb0 = jnp.concatenate([jnp.sin(args), jnp.cos(args)], axis=1)
        z = _mmT(temb0, pack_ref[_R_W1:_R_W1 + 512, :]) + b1_ref[...]
        t1 = 0.5 * z * (1.0 + jax.lax.erf(z * np.float32(1.0 / np.sqrt(2.0))))
        temb = (_mm(t1, pack_ref[_R_W2T:_R_W2T + 512, :])
                + pack_ref[_R_B2:_R_B2 + 1, :])
        gate = jax.nn.sigmoid(_mm(temb, pack_ref[_R_GWT:_R_GWT + HID, :])
                              + pack_ref[_R_GB2:_R_GB2 + 1, :])
        csl = (_mm(posl4_ref[...], pack_ref[_R_CW4:_R_CW4 + 4, :])
               + pack_ref[_R_CB:_R_CB + 1, :])
        h_lig = csl * gate + _mm(temb, pack_ref[_R_BWT:_R_BWT + HID, :])
        hl_ref[...] = _mm(h_lig, pack_ref[_R_GCNT:_R_GCNT + HID, :])

    lo = bounds_ref[i, 0]
    hi = bounds_ref[i, 1]

    acc_ref[...] = jnp.zeros_like(acc_ref)
    deg_ref[...] = jnp.zeros_like(deg_ref)

    posl = posl4_ref[pl.ds(i * TL, TL), :]
    sql = sql_ref[pl.ds(i * TL, TL), :]

    for c in range(NSC):
        @pl.when((lo <= c) & (c < hi))
        def _chunk(c=c):
            off = c * CS
            poss_c = poss4_ref[off:off + CS, :]
            hs_c = hs_ref[off:off + CS, :]
            sqs_c = sqs_ref[c:c + 1, :]
            cross = jax.lax.dot_general(posl, poss_c,
                                        (((1,), (1,)), ((), ())),
                                        precision=_PH,
                                        preferred_element_type=jnp.float32)
            d2 = sql + sqs_c - 2.0 * cross
            adj = (d2 < R2).astype(jnp.float32)
            acc_ref[...] += _mm(adj, hs_c)
            deg_ref[...] += jnp.sum(adj, axis=1, keepdims=True)

    dis = 1.0 / jnp.sqrt(1.0 + deg_ref[...])
    hl = hl_ref[pl.ds(i * TL, TL), :]
    out = acc_ref[...] * dis + hl * (dis * dis) + pack_ref[_R_GB:_R_GB + 1, :]
    pred_ref[...] = (_mmT(out, pack_ref[_R_WP:_R_WP + 3, :])
                     + pack_ref[_R_BP:_R_BP + 1, 0:3])


def kernel(surface_pos, init_ligand_pos, batch_surface, batch_ligand, time,
           surf_enc_W, surf_enc_b, time_W1, time_b1, time_W2, time_b2,
           csl_W, csl_b, csl_gate_W, csl_gate_b, csl_bias_W,
           gcn_W, gcn_b, pos_mlp_W, pos_mlp_b):
    n_surf = surface_pos.shape[0]
    n_lig = init_ligand_pos.shape[0]
    W = gcn_W[-1]
    b = gcn_b[-1]

    bs_w = surf_enc_W.T @ W.T
    bs_b = surf_enc_b @ W.T

    bs_i = jnp.pad(batch_surface.astype(jnp.int32), (0, NSP - n_surf),
                   constant_values=5)
    bl_i = jnp.pad(batch_ligand.astype(jnp.int32), (0, NLP - n_lig),
                   constant_values=4)

    pos_l4 = jnp.concatenate(
        [jnp.pad(init_ligand_pos, ((0, NLP - n_lig), (0, 0))),
         bl_i.astype(jnp.float32)[:, None] * BC], axis=1)
    pos_s4 = jnp.concatenate(
        [jnp.pad(surface_pos, ((0, NSP - n_surf), (0, 0))),
         bs_i.astype(jnp.float32)[:, None] * BC], axis=1)
    t_pad = jnp.pad(time, ((0, NLP - n_lig), (0, 0)))
    sqs = jnp.sum(pos_s4 * pos_s4, axis=1).reshape(NSC, CS)
    sql = jnp.sum(pos_l4 * pos_l4, axis=1)[:, None]

    bl_r = bl_i.reshape(NLP // TL, TL)
    start = jnp.sum(bs_i[None, :] < bl_r[:, 0][:, None], axis=1)
    end = jnp.sum(bs_i[None, :] <= bl_r[:, -1][:, None], axis=1)
    bounds = jnp.stack([start // CS, (end + CS - 1) // CS],
                       axis=1).astype(jnp.int32)

    z3 = jnp.zeros((1, HID), jnp.float32)
    pack = jnp.concatenate([
        time_W1,
        time_W2.T,
        csl_gate_W.T,
        csl_bias_W.T,
        W.T,
        csl_W.T, z3,
        bs_w, z3,
        bs_b[None, :],
        time_b2[None, :],
        csl_gate_b[None, :],
        csl_b[None, :],
        b[None, :],
        pos_mlp_W,
        jnp.pad(pos_mlp_b, (0, HID - 3))[None, :],
        jnp.zeros((_PACK_ROWS - 1425, HID), jnp.float32),
    ], axis=0)

    pred = pl.pallas_call(
        _fused_kernel,
        grid_spec=pltpu.PrefetchScalarGridSpec(
            num_scalar_prefetch=1,
            grid=(NLP // TL,),
            in_specs=[
                pl.BlockSpec((NLP, 4), lambda i, b_: (0, 0)),
                pl.BlockSpec((NLP, 1), lambda i, b_: (0, 0)),
                pl.BlockSpec((NLP, 1), lambda i, b_: (0, 0)),
                pl.BlockSpec((NSP, 4), lambda i, b_: (0, 0)),
                pl.BlockSpec((NSC, CS), lambda i, b_: (0, 0)),
                pl.BlockSpec((_PACK_ROWS, HID), lambda i, b_: (0, 0)),
                pl.BlockSpec((1, 512), lambda i, b_: (0, 0)),
            ],
            out_specs=pl.BlockSpec((TL, 3), lambda i, b_: (i, 0)),
            scratch_shapes=[pltpu.VMEM((NSP, HID), jnp.float32),
                            pltpu.VMEM((NLP, HID), jnp.float32),
                            pltpu.VMEM((TL, HID), jnp.float32),
                            pltpu.VMEM((TL, 1), jnp.float32)],
        ),
        out_shape=jax.ShapeDtypeStruct((NLP, 3), jnp.float32),
        compiler_params=pltpu.CompilerParams(
            dimension_semantics=("arbitrary",)),
    )(bounds, pos_l4, sql, t_pad, pos_s4, sqs, pack, time_b1[None, :])

    return pred[:n_lig]


# speedup vs baseline: 1.0349x; 1.0349x over previous
import jax
import jax.numpy as jnp
import numpy as np
from jax.experimental import pallas as pl
from jax.experimental.pallas import tpu as pltpu

_PH = jax.lax.Precision.HIGHEST

NLP = 2048
NSP = 8192
TL = 512
CS = 1024
NSC = NSP // CS
HID = 128
R2 = 3.5 * 3.5


def _mmT(x, w):
    return jax.lax.dot_general(x, w, (((1,), (1,)), ((), ())),
                               preferred_element_type=jnp.float32)


def _mm(x, w):
    return jax.lax.dot_general(x, w, (((1,), (0,)), ((), ())),
                               preferred_element_type=jnp.float32)


def _fused_kernel(bounds_ref,
                  sql_ref, bl_ref, posl_ref, posl_full_ref, t_full_ref,
                  sqs_ref, bs_ref, poss_ref,
                  bsw_ref, bsb_ref,
                  w1_ref, b1_ref, w2_ref, b2_ref,
                  gw_ref, gb2_ref, cw_ref, cb_ref, biasw_ref, gcnw_ref,
                  gb_ref, wp_ref, bp_ref,
                  pred_ref, hs_ref, hl_ref, acc_ref, deg_ref):
    i = pl.program_id(0)

    @pl.when(i == 0)
    def _features():
        hs_ref[...] = _mm(poss_ref[...], bsw_ref[...]) + bsb_ref[...]
        half = HID // 2
        emb = np.log(10000.0) / (half - 1)
        k = jax.lax.broadcasted_iota(jnp.int32, (1, half), 1).astype(
            jnp.float32)
        freqs = jnp.exp(k * (-emb))
        args = t_full_ref[...] * freqs
        temb0 = jnp.concatenate([jnp.sin(args), jnp.cos(args)], axis=1)
        z = _mmT(temb0, w1_ref[...]) + b1_ref[...]
        t1 = 0.5 * z * (1.0 + jax.lax.erf(z * np.float32(1.0 / np.sqrt(2.0))))
        temb = _mmT(t1, w2_ref[...]) + b2_ref[...]
        gate = jax.nn.sigmoid(_mmT(temb, gw_ref[...]) + gb2_ref[...])
        csl = _mmT(posl_full_ref[...], cw_ref[...]) + cb_ref[...]
        h_lig = csl * gate + _mmT(temb, biasw_ref[...])
        hl_ref[...] = _mmT(h_lig, gcnw_ref[...])

    lo = bounds_ref[i, 0]
    hi = bounds_ref[i, 1]

    acc_ref[...] = jnp.zeros_like(acc_ref)
    deg_ref[...] = jnp.zeros_like(deg_ref)

    sql = sql_ref[...]
    bl = bl_ref[...]
    posl = posl_ref[...]

    for c in range(NSC):
        @pl.when((lo <= c) & (c < hi))
        def _chunk(c=c):
            off = c * CS
            poss_c = poss_ref[off:off + CS, :]
            hs_c = hs_ref[off:off + CS, :]
            sqs_c = sqs_ref[c:c + 1, :]
            bs_c = bs_ref[c:c + 1, :]
            cross = jax.lax.dot_general(posl, poss_c,
                                        (((1,), (1,)), ((), ())),
                                        precision=_PH,
                                        preferred_element_type=jnp.float32)
            d2 = sql + sqs_c - 2.0 * cross
            adj = ((d2 < R2) & (bl == bs_c)).astype(jnp.float32)
            acc_ref[...] += _mm(adj, hs_c)
            deg_ref[...] += jnp.sum(adj, axis=1, keepdims=True)

    dis = 1.0 / jnp.sqrt(1.0 + deg_ref[...])
    hl = hl_ref[pl.ds(i * TL, TL), :]
    out = acc_ref[...] * dis + hl * (dis * dis) + gb_ref[...]
    pred_ref[...] = _mmT(out, wp_ref[...]) + bp_ref[...]


def kernel(surface_pos, init_ligand_pos, batch_surface, batch_ligand, time,
           surf_enc_W, surf_enc_b, time_W1, time_b1, time_W2, time_b2,
           csl_W, csl_b, csl_gate_W, csl_gate_b, csl_bias_W,
           gcn_W, gcn_b, pos_mlp_W, pos_mlp_b):
    n_surf = surface_pos.shape[0]
    n_lig = init_ligand_pos.shape[0]
    W = gcn_W[-1]
    b = gcn_b[-1]

    bs_w = surf_enc_W.T @ W.T
    bs_b = (surf_enc_b @ W.T)[None, :]

    pos_s = jnp.pad(surface_pos, ((0, NSP - n_surf), (0, 0)))
    pos_l = jnp.pad(init_ligand_pos, ((0, NLP - n_lig), (0, 0)))
    t_pad = jnp.pad(time, ((0, NLP - n_lig), (0, 0)))
    bs_i = jnp.pad(batch_surface.astype(jnp.int32), (0, NSP - n_surf),
                   constant_values=5)
    bl_i = jnp.pad(batch_ligand.astype(jnp.int32), (0, NLP - n_lig),
                   constant_values=4)
    bsf = bs_i.astype(jnp.float32).reshape(NSC, CS)
    blf = bl_i.astype(jnp.float32)[:, None]
    sqs = jnp.sum(pos_s * pos_s, axis=1).reshape(NSC, CS)
    sql = jnp.sum(pos_l * pos_l, axis=1)[:, None]

    bl_r = bl_i.reshape(NLP // TL, TL)
    start = jnp.sum(bs_i[None, :] < bl_r[:, 0][:, None], axis=1)
    end = jnp.sum(bs_i[None, :] <= bl_r[:, -1][:, None], axis=1)
    bounds = jnp.stack([start // CS, (end + CS - 1) // CS],
                       axis=1).astype(jnp.int32)

    pred = pl.pallas_call(
        _fused_kernel,
        grid_spec=pltpu.PrefetchScalarGridSpec(
            num_scalar_prefetch=1,
            grid=(NLP // TL,),
            in_specs=[
                pl.BlockSpec((TL, 1), lambda i, b_: (i, 0)),
                pl.BlockSpec((TL, 1), lambda i, b_: (i, 0)),
                pl.BlockSpec((TL, 3), lambda i, b_: (i, 0)),
                pl.BlockSpec((NLP, 3), lambda i, b_: (0, 0)),
                pl.BlockSpec((NLP, 1), lambda i, b_: (0, 0)),
                pl.BlockSpec((NSC, CS), lambda i, b_: (0, 0)),
                pl.BlockSpec((NSC, CS), lambda i, b_: (0, 0)),
                pl.BlockSpec((NSP, 3), lambda i, b_: (0, 0)),
                pl.BlockSpec((3, HID), lambda i, b_: (0, 0)),
                pl.BlockSpec((1, HID), lambda i, b_: (0, 0)),
                pl.BlockSpec((512, HID), lambda i, b_: (0, 0)),
                pl.BlockSpec((1, 512), lambda i, b_: (0, 0)),
                pl.BlockSpec((HID, 512), lambda i, b_: (0, 0)),
                pl.BlockSpec((1, HID), lambda i, b_: (0, 0)),
                pl.BlockSpec((HID, HID), lambda i, b_: (0, 0)),
                pl.BlockSpec((1, HID), lambda i, b_: (0, 0)),
                pl.BlockSpec((HID, 3), lambda i, b_: (0, 0)),
                pl.BlockSpec((1, HID), lambda i, b_: (0, 0)),
                pl.BlockSpec((HID, HID), lambda i, b_: (0, 0)),
                pl.BlockSpec((HID, HID), lambda i, b_: (0, 0)),
                pl.BlockSpec((1, HID), lambda i, b_: (0, 0)),
                pl.BlockSpec((3, HID), lambda i, b_: (0, 0)),
                pl.BlockSpec((1, 3), lambda i, b_: (0, 0)),
            ],
            out_specs=pl.BlockSpec((TL, 3), lambda i, b_: (i, 0)),
            scratch_shapes=[pltpu.VMEM((NSP, HID), jnp.float32),
                            pltpu.VMEM((NLP, HID), jnp.float32),
                            pltpu.VMEM((TL, HID), jnp.float32),
                            pltpu.VMEM((TL, 1), jnp.float32)],
        ),
        out_shape=jax.ShapeDtypeStruct((NLP, 3), jnp.float32),
        compiler_params=pltpu.CompilerParams(
            dimension_semantics=("arbitrary",)),
    )(bounds, sql, blf, pos_l, pos_l, t_pad, sqs, bsf, pos_s,
      bs_w, bs_b,
      time_W1, time_b1[None, :], time_W2, time_b2[None, :],
      csl_gate_W, csl_gate_b[None, :], csl_W, csl_b[None, :],
      csl_bias_W, W,
      b[None, :], pos_mlp_W, pos_mlp_b[None, :])

    return pred[:n_lig]


# speedup vs baseline: 1.1085x; 1.0712x over previous
import jax
import jax.numpy as jnp
import numpy as np
from jax.experimental import pallas as pl
from jax.experimental.pallas import tpu as pltpu

_PH = jax.lax.Precision.HIGHEST

NLP = 2048
NSP = 8192
TL = 256
CS = 1024
NSC = NSP // CS
HID = 128
R2 = 3.5 * 3.5


def _mmT(x, w):
    return jax.lax.dot_general(x, w, (((1,), (1,)), ((), ())),
                               preferred_element_type=jnp.float32)


def _mm(x, w):
    return jax.lax.dot_general(x, w, (((1,), (0,)), ((), ())),
                               preferred_element_type=jnp.float32)


def _fused_kernel(bounds_ref,
                  sql_ref, bl_ref, posl_ref, posl_full_ref, t_full_ref,
                  sqs_ref, bs_ref, poss_ref,
                  bsw_ref, bsb_ref,
                  w1_ref, b1_ref, w2_ref, b2_ref,
                  gw_ref, gb2_ref, cw_ref, cb_ref, biasw_ref, gcnw_ref,
                  gb_ref, wp_ref, bp_ref,
                  pred_ref, hs_ref, hl_ref, acc_ref, deg_ref):
    i = pl.program_id(0)

    @pl.when(i == 0)
    def _features():
        hs_ref[...] = _mm(poss_ref[...], bsw_ref[...]) + bsb_ref[...]
        half = HID // 2
        emb = np.log(10000.0) / (half - 1)
        k = jax.lax.broadcasted_iota(jnp.int32, (1, half), 1).astype(
            jnp.float32)
        freqs = jnp.exp(k * (-emb))
        args = t_full_ref[...] * freqs
        temb0 = jnp.concatenate([jnp.sin(args), jnp.cos(args)], axis=1)
        z = _mmT(temb0, w1_ref[...]) + b1_ref[...]
        t1 = 0.5 * z * (1.0 + jax.lax.erf(z * np.float32(1.0 / np.sqrt(2.0))))
        temb = _mmT(t1, w2_ref[...]) + b2_ref[...]
        gate = jax.nn.sigmoid(_mmT(temb, gw_ref[...]) + gb2_ref[...])
        csl = _mmT(posl_full_ref[...], cw_ref[...]) + cb_ref[...]
        h_lig = csl * gate + _mmT(temb, biasw_ref[...])
        hl_ref[...] = _mmT(h_lig, gcnw_ref[...])

    lo = bounds_ref[i, 0]
    hi = bounds_ref[i, 1]

    acc_ref[...] = jnp.zeros_like(acc_ref)
    deg_ref[...] = jnp.zeros_like(deg_ref)

    sql = sql_ref[...]
    bl = bl_ref[...]
    posl = posl_ref[...]

    for c in range(NSC):
        @pl.when((lo <= c) & (c < hi))
        def _chunk(c=c):
            off = c * CS
            poss_c = poss_ref[off:off + CS, :]
            hs_c = hs_ref[off:off + CS, :]
            sqs_c = sqs_ref[c:c + 1, :]
            bs_c = bs_ref[c:c + 1, :]
            cross = jax.lax.dot_general(posl, poss_c,
                                        (((1,), (1,)), ((), ())),
                                        precision=_PH,
                                        preferred_element_type=jnp.float32)
            d2 = sql + sqs_c - 2.0 * cross
            mask = (d2 < R2) & (bl == bs_c)
            adj = mask.astype(jnp.float32)
            acc_ref[...] += _mm(mask.astype(jnp.bfloat16),
                                hs_c.astype(jnp.bfloat16))
            deg_ref[...] += jnp.sum(adj, axis=1, keepdims=True)

    dis = 1.0 / jnp.sqrt(1.0 + deg_ref[...])
    hl = hl_ref[pl.ds(i * TL, TL), :]
    out = acc_ref[...] * dis + hl * (dis * dis) + gb_ref[...]
    pred_ref[...] = _mmT(out, wp_ref[...]) + bp_ref[...]


def kernel(surface_pos, init_ligand_pos, batch_surface, batch_ligand, time,
           surf_enc_W, surf_enc_b, time_W1, time_b1, time_W2, time_b2,
           csl_W, csl_b, csl_gate_W, csl_gate_b, csl_bias_W,
           gcn_W, gcn_b, pos_mlp_W, pos_mlp_b):
    n_surf = surface_pos.shape[0]
    n_lig = init_ligand_pos.shape[0]
    W = gcn_W[-1]
    b = gcn_b[-1]

    bs_w = surf_enc_W.T @ W.T
    bs_b = (surf_enc_b @ W.T)[None, :]

    pos_s = jnp.pad(surface_pos, ((0, NSP - n_surf), (0, 0)))
    pos_l = jnp.pad(init_ligand_pos, ((0, NLP - n_lig), (0, 0)))
    t_pad = jnp.pad(time, ((0, NLP - n_lig), (0, 0)))
    bs_i = jnp.pad(batch_surface.astype(jnp.int32), (0, NSP - n_surf),
                   constant_values=5)
    bl_i = jnp.pad(batch_ligand.astype(jnp.int32), (0, NLP - n_lig),
                   constant_values=4)
    bsf = bs_i.astype(jnp.float32).reshape(NSC, CS)
    blf = bl_i.astype(jnp.float32)[:, None]
    sqs = jnp.sum(pos_s * pos_s, axis=1).reshape(NSC, CS)
    sql = jnp.sum(pos_l * pos_l, axis=1)[:, None]

    bl_r = bl_i.reshape(NLP // TL, TL)
    start = jnp.sum(bs_i[None, :] < bl_r[:, 0][:, None], axis=1)
    end = jnp.sum(bs_i[None, :] <= bl_r[:, -1][:, None], axis=1)
    bounds = jnp.stack([start // CS, (end + CS - 1) // CS],
                       axis=1).astype(jnp.int32)

    pred = pl.pallas_call(
        _fused_kernel,
        grid_spec=pltpu.PrefetchScalarGridSpec(
            num_scalar_prefetch=1,
            grid=(NLP // TL,),
            in_specs=[
                pl.BlockSpec((TL, 1), lambda i, b_: (i, 0)),
                pl.BlockSpec((TL, 1), lambda i, b_: (i, 0)),
                pl.BlockSpec((TL, 3), lambda i, b_: (i, 0)),
                pl.BlockSpec((NLP, 3), lambda i, b_: (0, 0)),
                pl.BlockSpec((NLP, 1), lambda i, b_: (0, 0)),
                pl.BlockSpec((NSC, CS), lambda i, b_: (0, 0)),
                pl.BlockSpec((NSC, CS), lambda i, b_: (0, 0)),
                pl.BlockSpec((NSP, 3), lambda i, b_: (0, 0)),
                pl.BlockSpec((3, HID), lambda i, b_: (0, 0)),
                pl.BlockSpec((1, HID), lambda i, b_: (0, 0)),
                pl.BlockSpec((512, HID), lambda i, b_: (0, 0)),
                pl.BlockSpec((1, 512), lambda i, b_: (0, 0)),
                pl.BlockSpec((HID, 512), lambda i, b_: (0, 0)),
                pl.BlockSpec((1, HID), lambda i, b_: (0, 0)),
                pl.BlockSpec((HID, HID), lambda i, b_: (0, 0)),
                pl.BlockSpec((1, HID), lambda i, b_: (0, 0)),
                pl.BlockSpec((HID, 3), lambda i, b_: (0, 0)),
                pl.BlockSpec((1, HID), lambda i, b_: (0, 0)),
                pl.BlockSpec((HID, HID), lambda i, b_: (0, 0)),
                pl.BlockSpec((HID, HID), lambda i, b_: (0, 0)),
                pl.BlockSpec((1, HID), lambda i, b_: (0, 0)),
                pl.BlockSpec((3, HID), lambda i, b_: (0, 0)),
                pl.BlockSpec((1, 3), lambda i, b_: (0, 0)),
            ],
            out_specs=pl.BlockSpec((TL, 3), lambda i, b_: (i, 0)),
            scratch_shapes=[pltpu.VMEM((NSP, HID), jnp.float32),
                            pltpu.VMEM((NLP, HID), jnp.float32),
                            pltpu.VMEM((TL, HID), jnp.float32),
                            pltpu.VMEM((TL, 1), jnp.float32)],
        ),
        out_shape=jax.ShapeDtypeStruct((NLP, 3), jnp.float32),
        compiler_params=pltpu.CompilerParams(
            dimension_semantics=("arbitrary",)),
    )(bounds, sql, blf, pos_l, pos_l, t_pad, sqs, bsf, pos_s,
      bs_w, bs_b,
      time_W1, time_b1[None, :], time_W2, time_b2[None, :],
      csl_gate_W, csl_gate_b[None, :], csl_W, csl_b[None, :],
      csl_bias_W, W,
      b[None, :], pos_mlp_W, pos_mlp_b[None, :])

    return pred[:n_lig]
